# baseline (speedup 1.0000x reference)
import functools

import jax
import jax.numpy as jnp
import numpy as np
from jax import lax
from jax.experimental import pallas as pl
from jax.experimental.pallas import tpu as pltpu
from jax.experimental.pallas import tpu_sc as plsc

_CAP = 0.5
_INT_MIN = np.int32(-2147483648)


def _float_keys(w):
    i = jax.lax.bitcast_convert_type(w, jnp.int32)
    return jnp.where(i >= 0, i, _INT_MIN - i)


def _router_kernel(x_ref, wr_ref, br_ref, out_ref):
    wcol = wr_ref[...].astype(jnp.bfloat16).astype(jnp.float32)
    xa = x_ref[0].astype(jnp.bfloat16).astype(jnp.float32)
    out_ref[0, :] = jnp.sum(xa * wcol, axis=0) + br_ref[0, 0]


def _thresh_kernel(w0_ref, w1_ref, thr_ref, *, k):
    keys = jnp.stack([_float_keys(w0_ref[...]),
                      _float_keys(w1_ref[...])])
    cnt0 = jnp.sum((keys >= 0).astype(jnp.int32), axis=(1, 2), keepdims=True)
    cand = jnp.where(cnt0 >= k, np.int32(0), _INT_MIN)
    for bit in range(30, -1, -1):
        trial = cand | np.int32(1 << bit)
        cnt = jnp.sum((keys >= trial).astype(jnp.int32), axis=(1, 2),
                      keepdims=True)
        cand = jnp.where(cnt >= k, trial, cand)
    ival = jnp.where(cand >= 0, cand, _INT_MIN - cand)
    thr_ref[...] = jax.lax.bitcast_convert_type(ival, jnp.float32)


def _out_kernel(x_ref, w0_ref, w1_ref, thr_ref, wb_ref, bb_ref, out_ref):
    wb = wb_ref[...].astype(jnp.bfloat16)
    bb = bb_ref[...]
    lrefs = [w0_ref, w1_ref]
    for a in range(len(lrefs)):
        y = jnp.dot(wb, x_ref[a].astype(jnp.bfloat16),
                    preferred_element_type=jnp.float32) + bb
        mask = lrefs[a][...] > thr_ref[a]
        out_ref[a] = jnp.where(mask, y, x_ref[a])


def _sc_router_body(x_hbm, wr_hbm, br_hbm, out_hbm,
                    wr_v, br_v, buf, ltile, *, c, nc, nt_base, extra):
    wid = lax.axis_index("s") * nc + lax.axis_index("c")
    base = wid * nt_base + jnp.minimum(wid, extra)
    ntiles = nt_base + jnp.where(wid < extra, 1, 0)

    pltpu.sync_copy(wr_hbm, wr_v)
    pltpu.sync_copy(br_hbm, br_v)
    br = br_v[...][0]

    def tile_step(i, carry):
        @pl.when(i < ntiles)
        def _do():
            toff = (base + i) * 128
            pltpu.sync_copy(x_hbm.at[1, :, pl.ds(toff, 128)], buf)

            def body(c16, accs):
                wvec = wr_v[pl.ds(c16 * 16, 16)]
                accs = list(accs)
                for j in range(16):
                    w = wvec[j]
                    cc = c16 * 16 + j
                    for g in range(8):
                        xb = buf[cc, pl.ds(g * 16, 16)]
                        bi = jax.lax.bitcast_convert_type(xb, jnp.int32)
                        bi = (bi + np.int32(0x7FFF)
                              + (jax.lax.shift_right_logical(bi, 16)
                                 & np.int32(1))) & np.int32(-65536)
                        xb = jax.lax.bitcast_convert_type(bi, jnp.float32)
                        accs[g] = accs[g] + xb * w
                return tuple(accs)

            accs = lax.fori_loop(
                0, c // 16, body, tuple(jnp.zeros((16,), jnp.float32)
                                        for _ in range(8)))
            for g in range(8):
                ltile[pl.ds(g * 16, 16)] = accs[g] + br
            pltpu.sync_copy(ltile, out_hbm.at[pl.ds(toff, 128)])
        return carry

    lax.fori_loop(0, nt_base + (1 if extra else 0), tile_step, jnp.int32(0))


def _sc_router(xf, wr_rounded, br16, T, c):
    info = plsc.get_sparse_core_info()
    nc, ns = info.num_cores, info.num_subcores
    nw = nc * ns
    ntile = T // 128
    nt_base, extra = divmod(ntile, nw)

    run = pl.kernel(
        functools.partial(_sc_router_body, c=c, nc=nc, nt_base=nt_base,
                          extra=extra),
        mesh=plsc.VectorSubcoreMesh(core_axis_name="c", subcore_axis_name="s"),
        out_type=jax.ShapeDtypeStruct((T,), jnp.float32),
        scratch_types=[
            pltpu.VMEM((c,), jnp.float32),
            pltpu.VMEM((16,), jnp.float32),
            pltpu.VMEM((c, 128), jnp.float32),
            pltpu.VMEM((128,), jnp.float32),
        ],
    )
    return run(xf, wr_rounded, br16)


def kernel(x, w_r, b_r, W_b, b_b):
    nb, c, s1, d1 = x.shape
    T = s1 * d1
    k = int(_CAP * T)
    xf = x.reshape(nb, c, T)

    sblk = T
    for cand in (7168, 4096, 3584, 3136, 2048, 1792, 1024, 512):
        if T % cand == 0:
            sblk = cand
            break
    nblk = T // sblk

    wr2 = w_r.reshape(c, 1)
    br2 = b_r.reshape(1, 1)
    bb2 = b_b.reshape(c, 1)

    logits0 = pl.pallas_call(
        _router_kernel,
        grid=(nblk,),
        in_specs=[
            pl.BlockSpec((1, c, sblk), lambda i: (0, 0, i)),
            pl.BlockSpec((c, 1), lambda i: (0, 0)),
            pl.BlockSpec((1, 1), lambda i: (0, 0)),
        ],
        out_specs=pl.BlockSpec((1, sblk), lambda i: (0, i)),
        out_shape=jax.ShapeDtypeStruct((1, T), jnp.float32),
    )(xf, wr2, br2)

    wr_rounded = w_r.astype(jnp.bfloat16).astype(jnp.float32)
    br8 = jnp.pad(b_r, (0, 15))
    logits1 = _sc_router(xf, wr_rounded, br8, T, c).reshape(1, T)

    l0 = logits0.reshape(T // 128, 128)
    l1 = logits1.reshape(T // 128, 128)
    thr = pl.pallas_call(
        functools.partial(_thresh_kernel, k=k),
        in_specs=[pl.BlockSpec(l0.shape, lambda: (0, 0)),
                  pl.BlockSpec(l1.shape, lambda: (0, 0))],
        out_specs=pl.BlockSpec((nb, 1, 1), lambda: (0, 0, 0)),
        out_shape=jax.ShapeDtypeStruct((nb, 1, 1), jnp.float32),
    )(l0, l1)

    out = pl.pallas_call(
        _out_kernel,
        grid=(nblk,),
        in_specs=[
            pl.BlockSpec((nb, c, sblk), lambda i: (0, 0, i)),
            pl.BlockSpec((1, sblk), lambda i: (0, i)),
            pl.BlockSpec((1, sblk), lambda i: (0, i)),
            pl.BlockSpec((nb, 1, 1), lambda i: (0, 0, 0)),
            pl.BlockSpec((c, c), lambda i: (0, 0)),
            pl.BlockSpec((c, 1), lambda i: (0, 0)),
        ],
        out_specs=pl.BlockSpec((nb, c, sblk), lambda i: (0, 0, i)),
        out_shape=jax.ShapeDtypeStruct((nb, c, T), jnp.float32),
    )(xf, logits0, logits1, thr, W_b, bb2)

    return out.reshape(nb, c, s1, d1)
